# baseline (speedup 1.0000x reference)
import functools

import jax
import jax.numpy as jnp
from jax import lax
from jax.experimental import pallas as pl
from jax.experimental.pallas import tpu as pltpu
from jax.experimental.pallas import tpu_sc as plsc

NC, NS, L = 2, 16, 16
NW = NC * NS
CW = 128
T = 64
NBUF = 4


def _sc_call(N, H, NBINS):
    SPT = N // NS
    CG = H // CW
    TS = NW // CG
    NPS = N // TS
    CHUNKS = NPS // T
    R = 2 * NBINS
    CH = CW // L

    mesh = plsc.VectorSubcoreMesh(core_axis_name="c", subcore_axis_name="s")

    @functools.partial(
        pl.kernel,
        out_type=jax.ShapeDtypeStruct((N, H), jnp.float32),
        mesh=mesh,
        compiler_params=pltpu.CompilerParams(needs_layout_passes=False),
        scratch_types=[
            pltpu.VMEM((R, CW // 2), jnp.int32),
            pltpu.VMEM((NBUF, T, CW), jnp.float32),
            pltpu.VMEM((NBUF, T), jnp.int32),
            pltpu.VMEM((NBUF, T), jnp.int32),
            pltpu.VMEM((SPT,), jnp.float32),
            pltpu.VMEM((SPT,), jnp.float32),
            pltpu.VMEM((SPT,), jnp.int32),
            pltpu.VMEM((SPT,), jnp.int32),
            pltpu.VMEM((2 * NBINS,), jnp.float32),
            pltpu.VMEM_SHARED((N,), jnp.int32),
            pltpu.VMEM_SHARED((N,), jnp.int32),
            pltpu.SemaphoreType.DMA,
            pltpu.SemaphoreType.DMA,
            pltpu.SemaphoreType.DMA,
            pltpu.SemaphoreType.DMA,
            pltpu.SemaphoreType.DMA,
            pltpu.SemaphoreType.DMA,
            pltpu.SemaphoreType.DMA,
            pltpu.SemaphoreType.DMA,
            pltpu.SemaphoreType.DMA,
            pltpu.SemaphoreType.DMA,
            pltpu.SemaphoreType.DMA,
            pltpu.SemaphoreType.DMA,
        ],
    )
    def body(enc_hbm, pv_hbm, ev_hbm, ctab_hbm, bnd_hbm, out_hbm,
             tab, out_b, pb_b, eb_b, pvals, evals, pidx, eidx, bnds,
             pidx_sh, eidx_sh,
             se0, se1, se2, se3, si0, si1, si2, si3, sw0, sw1, sw2, sw3):
        cid = lax.axis_index("c")
        sid = lax.axis_index("s")
        wid = cid * NS + sid
        gcol = (wid % CG) * CW
        tok0 = (wid // CG) * NPS
        se = (se0, se1, se2, se3)
        si = (si0, si1, si2, si3)
        sw = (sw0, sw1, sw2, sw3)

        pltpu.sync_copy(bnd_hbm, bnds)
        qbase = sid * SPT
        pltpu.sync_copy(pv_hbm.at[pl.ds(qbase, SPT)], pvals)
        pltpu.sync_copy(ev_hbm.at[pl.ds(qbase, SPT)], evals)

        @plsc.parallel_loop(0, SPT // L)
        def _search(j):
            sl = pl.ds(j * L, L)
            for vals_ref, idx_ref, base_bin in ((pvals, pidx, 0),
                                                (evals, eidx, NBINS)):
                v = vals_ref[sl]
                curr = jnp.zeros((L,), jnp.int32)
                step = NBINS // 2
                while step >= 1:
                    probe = plsc.load_gather(bnds, [curr + (base_bin + step - 1)])
                    curr = jnp.where(probe < v, curr + step, curr)
                    step //= 2
                idx_ref[sl] = curr + base_bin

        pltpu.sync_copy(pidx, pidx_sh.at[pl.ds(qbase, SPT)])
        pltpu.sync_copy(eidx, eidx_sh.at[pl.ds(qbase, SPT)])

        pltpu.sync_copy(ctab_hbm.at[wid % CG], tab)
        plsc.subcore_barrier()

        def issue(c, p):
            base = tok0 + c * T
            pltpu.async_copy(enc_hbm.at[pl.ds(base, T), pl.ds(gcol, CW)],
                             out_b.at[p], se[p])
            pltpu.async_copy(pidx_sh.at[pl.ds(base, T)], pb_b.at[p], si[p])
            pltpu.async_copy(eidx_sh.at[pl.ds(base, T)], eb_b.at[p], si[p])

        def wait_wb(p):
            pltpu.make_async_copy(out_b.at[p],
                                  out_hbm.at[pl.ds(tok0, T), pl.ds(gcol, CW)],
                                  sw[p]).wait()

        def finish(c, p):
            base = tok0 + c * T
            pltpu.make_async_copy(enc_hbm.at[pl.ds(base, T), pl.ds(gcol, CW)],
                                  out_b.at[p], se[p]).wait()
            pltpu.make_async_copy(pidx_sh.at[pl.ds(base, T)], pb_b.at[p],
                                  si[p]).wait()
            pltpu.make_async_copy(eidx_sh.at[pl.ds(base, T)], eb_b.at[p],
                                  si[p]).wait()

            @plsc.parallel_loop(0, T // L)
            def _row(j):
                t0 = j * L
                rpv = pb_b[p, pl.ds(t0, L)]
                rev = eb_b[p, pl.ds(t0, L)]
                for k in range(L):
                    for h2 in range(CW // (2 * L)):
                        sl = pl.ds(h2 * L, L)
                        pa, pb = plsc.unpack(
                            plsc.bitcast(tab[rpv[k], sl], jnp.bfloat16),
                            format=plsc.PackFormat.INTERLEAVED)
                        ea, eb = plsc.unpack(
                            plsc.bitcast(tab[rev[k], sl], jnp.bfloat16),
                            format=plsc.PackFormat.INTERLEAVED)
                        plsc.addupdate(
                            out_b.at[p, t0 + k, pl.ds(h2 * 2 * L, L)], pa + ea)
                        plsc.addupdate(
                            out_b.at[p, t0 + k, pl.ds(h2 * 2 * L + L, L)],
                            pb + eb)

            pltpu.async_copy(out_b.at[p],
                             out_hbm.at[pl.ds(base, T), pl.ds(gcol, CW)],
                             sw[p])

        issue(0, 0)
        issue(1, 1)

        @pl.loop(0, CHUNKS, step=NBUF)
        def _main(cc):
            for q in range(NBUF):
                c = cc + q
                finish(c, q)
                pnext = (q + 2) % NBUF
                if q < 2:
                    @pl.when(cc > 0)
                    def _():
                        wait_wb(pnext)
                else:
                    wait_wb(pnext)

                @pl.when(c + 2 < CHUNKS)
                def _():
                    issue(c + 2, pnext)

        wait_wb((CHUNKS - 2) % NBUF)
        wait_wb((CHUNKS - 1) % NBUF)

    return body


def kernel(encoder_output, pitch_target, energy_target, pitch_table, energy_table):
    B, S, H = encoder_output.shape
    N = B * S
    NBINS = pitch_table.shape[0]
    enc = encoder_output.reshape(N, H)
    pv = pitch_target.reshape(N)
    ev = energy_target.reshape(N)
    ctab = jnp.concatenate([pitch_table, energy_table], axis=0)
    R = 2 * NBINS
    CG = H // CW
    ctab = (ctab.astype(jnp.bfloat16)
            .reshape(R, CG, CW // 32, 2, 16)
            .transpose(1, 0, 2, 4, 3)
            .reshape(CG, R, CW // 2, 2))
    ctab = jax.lax.bitcast_convert_type(ctab, jnp.int32)
    bnds = jnp.concatenate([
        jnp.linspace(50.0, 400.0, NBINS),
        jnp.linspace(0.0, 1.0, NBINS),
    ])
    out = _sc_call(N, H, NBINS)(enc, pv, ev, ctab, bnds)
    return out.reshape(B, S, H)


# speedup vs baseline: 1.2770x; 1.2770x over previous
import functools

import jax
import jax.numpy as jnp
from jax import lax
from jax.experimental import pallas as pl
from jax.experimental.pallas import tpu as pltpu
from jax.experimental.pallas import tpu_sc as plsc

NC, NS, L = 2, 16, 16
NW = NC * NS
CW = 128
T = 128
NBUF = 3


def _sc_call(N, H, NBINS):
    SPT = N // NS
    CG = H // CW
    TS = NW // CG
    NPS = N // TS
    CHUNKS = NPS // T
    R = 2 * NBINS
    CH = CW // L

    mesh = plsc.VectorSubcoreMesh(core_axis_name="c", subcore_axis_name="s")

    @functools.partial(
        pl.kernel,
        out_type=jax.ShapeDtypeStruct((N, H), jnp.float32),
        mesh=mesh,
        compiler_params=pltpu.CompilerParams(needs_layout_passes=False),
        scratch_types=[
            pltpu.VMEM((R, CW // 2), jnp.int32),
            pltpu.VMEM((NBUF, T, CW), jnp.float32),
            pltpu.VMEM((NBUF, T), jnp.int32),
            pltpu.VMEM((SPT,), jnp.float32),
            pltpu.VMEM((SPT,), jnp.float32),
            pltpu.VMEM((SPT,), jnp.int32),
            pltpu.VMEM((2 * NBINS,), jnp.float32),
            pltpu.VMEM_SHARED((N,), jnp.int32),
            pltpu.SemaphoreType.DMA,
            pltpu.SemaphoreType.DMA,
            pltpu.SemaphoreType.DMA,
            pltpu.SemaphoreType.DMA,
            pltpu.SemaphoreType.DMA,
            pltpu.SemaphoreType.DMA,
            pltpu.SemaphoreType.DMA,
            pltpu.SemaphoreType.DMA,
            pltpu.SemaphoreType.DMA,
        ],
    )
    def body(enc_hbm, pv_hbm, ev_hbm, ctab_hbm, bnd_hbm, out_hbm,
             tab, out_b, cb_b, pvals, evals, cidx, bnds, cidx_sh,
             se0, se1, se2, si0, si1, si2, sw0, sw1, sw2):
        cid = lax.axis_index("c")
        sid = lax.axis_index("s")
        wid = cid * NS + sid
        gcol = (wid % CG) * CW
        tok0 = (wid // CG) * NPS
        se = (se0, se1, se2)
        si = (si0, si1, si2)
        sw = (sw0, sw1, sw2)

        pltpu.sync_copy(bnd_hbm, bnds)
        qbase = sid * SPT
        pltpu.sync_copy(pv_hbm.at[pl.ds(qbase, SPT)], pvals)
        pltpu.sync_copy(ev_hbm.at[pl.ds(qbase, SPT)], evals)

        @plsc.parallel_loop(0, SPT // L)
        def _search(j):
            sl = pl.ds(j * L, L)
            packed = jnp.zeros((L,), jnp.int32)
            for vals_ref, base_bin, shift in ((pvals, 0, 0),
                                              (evals, NBINS, 16)):
                v = vals_ref[sl]
                curr = jnp.zeros((L,), jnp.int32)
                step = NBINS // 2
                while step >= 1:
                    probe = plsc.load_gather(bnds, [curr + (base_bin + step - 1)])
                    curr = jnp.where(probe < v, curr + step, curr)
                    step //= 2
                packed = packed | ((curr + base_bin) << shift)
            cidx[sl] = packed

        pltpu.sync_copy(cidx, cidx_sh.at[pl.ds(qbase, SPT)])

        pltpu.sync_copy(ctab_hbm.at[wid % CG], tab)
        plsc.subcore_barrier()

        def issue(c, p):
            base = tok0 + c * T
            pltpu.async_copy(enc_hbm.at[pl.ds(base, T), pl.ds(gcol, CW)],
                             out_b.at[p], se[p])
            pltpu.async_copy(cidx_sh.at[pl.ds(base, T)], cb_b.at[p], si[p])

        def wait_wb(p):
            pltpu.make_async_copy(out_b.at[p],
                                  out_hbm.at[pl.ds(tok0, T), pl.ds(gcol, CW)],
                                  sw[p]).wait()

        def finish(c, p):
            base = tok0 + c * T
            pltpu.make_async_copy(enc_hbm.at[pl.ds(base, T), pl.ds(gcol, CW)],
                                  out_b.at[p], se[p]).wait()
            pltpu.make_async_copy(cidx_sh.at[pl.ds(base, T)], cb_b.at[p],
                                  si[p]).wait()

            @plsc.parallel_loop(0, T // L)
            def _row(j):
                t0 = j * L
                cv = cb_b[p, pl.ds(t0, L)]
                rpv = cv & 0xFFFF
                rev = cv >> 16
                for k in range(L):
                    for h2 in range(CW // (2 * L)):
                        sl = pl.ds(h2 * L, L)
                        pa, pb = plsc.unpack(
                            plsc.bitcast(tab[rpv[k], sl], jnp.bfloat16),
                            format=plsc.PackFormat.INTERLEAVED)
                        ea, eb = plsc.unpack(
                            plsc.bitcast(tab[rev[k], sl], jnp.bfloat16),
                            format=plsc.PackFormat.INTERLEAVED)
                        plsc.addupdate(
                            out_b.at[p, t0 + k, pl.ds(h2 * 2 * L, L)], pa + ea)
                        plsc.addupdate(
                            out_b.at[p, t0 + k, pl.ds(h2 * 2 * L + L, L)],
                            pb + eb)

            pltpu.async_copy(out_b.at[p],
                             out_hbm.at[pl.ds(base, T), pl.ds(gcol, CW)],
                             sw[p])

        issue(0, 0)
        issue(1, 1)

        @pl.loop(0, CHUNKS - 2, step=NBUF)
        def _main(cc):
            for q in range(NBUF):
                c = cc + q
                finish(c, q)
                pnext = (q + 2) % NBUF
                if q == 0:
                    @pl.when(cc > 0)
                    def _():
                        wait_wb(pnext)
                else:
                    wait_wb(pnext)
                issue(c + 2, pnext)

        finish(CHUNKS - 2, (CHUNKS - 2) % NBUF)
        wait_wb((CHUNKS - 3) % NBUF)
        finish(CHUNKS - 1, (CHUNKS - 1) % NBUF)
        wait_wb((CHUNKS - 2) % NBUF)
        wait_wb((CHUNKS - 1) % NBUF)

    return body


def kernel(encoder_output, pitch_target, energy_target, pitch_table, energy_table):
    B, S, H = encoder_output.shape
    N = B * S
    NBINS = pitch_table.shape[0]
    enc = encoder_output.reshape(N, H)
    pv = pitch_target.reshape(N)
    ev = energy_target.reshape(N)
    ctab = jnp.concatenate([pitch_table, energy_table], axis=0)
    R = 2 * NBINS
    CG = H // CW
    ctab = (ctab.astype(jnp.bfloat16)
            .reshape(R, CG, CW // 32, 2, 16)
            .transpose(1, 0, 2, 4, 3)
            .reshape(CG, R, CW // 2, 2))
    ctab = jax.lax.bitcast_convert_type(ctab, jnp.int32)
    bnds = jnp.concatenate([
        jnp.linspace(50.0, 400.0, NBINS),
        jnp.linspace(0.0, 1.0, NBINS),
    ])
    out = _sc_call(N, H, NBINS)(enc, pv, ev, ctab, bnds)
    return out.reshape(B, S, H)


# speedup vs baseline: 1.3008x; 1.0186x over previous
import functools

import jax
import jax.numpy as jnp
from jax import lax
from jax.experimental import pallas as pl
from jax.experimental.pallas import tpu as pltpu
from jax.experimental.pallas import tpu_sc as plsc

NC, NS, L = 2, 16, 16
NW = NC * NS
CW = 128
T = 128
NBUF = 3


def _sc_call(N, H, NBINS):
    SPT = N // NS
    CG = H // CW
    TS = NW // CG
    NPS = N // TS
    CHUNKS = NPS // T
    R = 2 * NBINS
    CH = CW // L

    mesh = plsc.VectorSubcoreMesh(core_axis_name="c", subcore_axis_name="s")

    @functools.partial(
        pl.kernel,
        out_type=jax.ShapeDtypeStruct((N, H), jnp.float32),
        mesh=mesh,
        compiler_params=pltpu.CompilerParams(needs_layout_passes=False),
        scratch_types=[
            pltpu.VMEM((R, CW // 2), jnp.int32),
            pltpu.VMEM((NBUF, T, CW), jnp.float32),
            pltpu.VMEM((NBUF, T), jnp.int32),
            pltpu.VMEM((SPT,), jnp.float32),
            pltpu.VMEM((SPT,), jnp.float32),
            pltpu.VMEM((SPT,), jnp.int32),
            pltpu.VMEM((2 * NBINS,), jnp.float32),
            pltpu.VMEM_SHARED((N,), jnp.int32),
            pltpu.SemaphoreType.DMA,
            pltpu.SemaphoreType.DMA,
            pltpu.SemaphoreType.DMA,
            pltpu.SemaphoreType.DMA,
            pltpu.SemaphoreType.DMA,
            pltpu.SemaphoreType.DMA,
            pltpu.SemaphoreType.DMA,
            pltpu.SemaphoreType.DMA,
            pltpu.SemaphoreType.DMA,
            pltpu.SemaphoreType.DMA,
        ],
    )
    def body(enc_hbm, pv_hbm, ev_hbm, ctab_hbm, bnd_hbm, out_hbm,
             tab, out_b, cb_b, pvals, evals, cidx, bnds, cidx_sh,
             se0, se1, se2, si0, si1, si2, sw0, sw1, sw2, st):
        cid = lax.axis_index("c")
        sid = lax.axis_index("s")
        wid = cid * NS + sid
        gcol = (wid % CG) * CW
        tok0 = (wid // CG) * NPS
        se = (se0, se1, se2)
        si = (si0, si1, si2)
        sw = (sw0, sw1, sw2)

        def issue_enc(c, p):
            base = tok0 + c * T
            pltpu.async_copy(enc_hbm.at[pl.ds(base, T), pl.ds(gcol, CW)],
                             out_b.at[p], se[p])

        tab_cp = pltpu.async_copy(ctab_hbm.at[wid % CG], tab, st)
        issue_enc(0, 0)
        issue_enc(1, 1)

        pltpu.sync_copy(bnd_hbm, bnds)
        qbase = sid * SPT
        pltpu.sync_copy(pv_hbm.at[pl.ds(qbase, SPT)], pvals)
        pltpu.sync_copy(ev_hbm.at[pl.ds(qbase, SPT)], evals)

        @plsc.parallel_loop(0, SPT // L)
        def _search(j):
            sl = pl.ds(j * L, L)
            packed = jnp.zeros((L,), jnp.int32)
            for vals_ref, base_bin, shift in ((pvals, 0, 0),
                                              (evals, NBINS, 16)):
                v = vals_ref[sl]
                curr = jnp.zeros((L,), jnp.int32)
                step = NBINS // 2
                while step >= 1:
                    probe = plsc.load_gather(bnds, [curr + (base_bin + step - 1)])
                    curr = jnp.where(probe < v, curr + step, curr)
                    step //= 2
                packed = packed | ((curr + base_bin) << shift)
            cidx[sl] = packed

        pltpu.sync_copy(cidx, cidx_sh.at[pl.ds(qbase, SPT)])

        tab_cp.wait()
        plsc.subcore_barrier()

        def issue_bins(c, p):
            base = tok0 + c * T
            pltpu.async_copy(cidx_sh.at[pl.ds(base, T)], cb_b.at[p], si[p])

        def issue(c, p):
            issue_enc(c, p)
            issue_bins(c, p)

        def wait_wb(p):
            pltpu.make_async_copy(out_b.at[p],
                                  out_hbm.at[pl.ds(tok0, T), pl.ds(gcol, CW)],
                                  sw[p]).wait()

        def finish(c, p):
            base = tok0 + c * T
            pltpu.make_async_copy(enc_hbm.at[pl.ds(base, T), pl.ds(gcol, CW)],
                                  out_b.at[p], se[p]).wait()
            pltpu.make_async_copy(cidx_sh.at[pl.ds(base, T)], cb_b.at[p],
                                  si[p]).wait()

            @plsc.parallel_loop(0, T // L)
            def _row(j):
                t0 = j * L
                cv = cb_b[p, pl.ds(t0, L)]
                rpv = cv & 0xFFFF
                rev = cv >> 16
                for k in range(L):
                    for h2 in range(CW // (2 * L)):
                        sl = pl.ds(h2 * L, L)
                        pa, pb = plsc.unpack(
                            plsc.bitcast(tab[rpv[k], sl], jnp.bfloat16),
                            format=plsc.PackFormat.INTERLEAVED)
                        ea, eb = plsc.unpack(
                            plsc.bitcast(tab[rev[k], sl], jnp.bfloat16),
                            format=plsc.PackFormat.INTERLEAVED)
                        plsc.addupdate(
                            out_b.at[p, t0 + k, pl.ds(h2 * 2 * L, L)], pa + ea)
                        plsc.addupdate(
                            out_b.at[p, t0 + k, pl.ds(h2 * 2 * L + L, L)],
                            pb + eb)

            pltpu.async_copy(out_b.at[p],
                             out_hbm.at[pl.ds(base, T), pl.ds(gcol, CW)],
                             sw[p])

        issue_bins(0, 0)
        issue_bins(1, 1)

        @pl.loop(0, CHUNKS - 2, step=NBUF)
        def _main(cc):
            for q in range(NBUF):
                c = cc + q
                finish(c, q)
                pnext = (q + 2) % NBUF
                if q == 0:
                    @pl.when(cc > 0)
                    def _():
                        wait_wb(pnext)
                else:
                    wait_wb(pnext)
                issue(c + 2, pnext)

        finish(CHUNKS - 2, (CHUNKS - 2) % NBUF)
        wait_wb((CHUNKS - 3) % NBUF)
        finish(CHUNKS - 1, (CHUNKS - 1) % NBUF)
        wait_wb((CHUNKS - 2) % NBUF)
        wait_wb((CHUNKS - 1) % NBUF)

    return body


def kernel(encoder_output, pitch_target, energy_target, pitch_table, energy_table):
    B, S, H = encoder_output.shape
    N = B * S
    NBINS = pitch_table.shape[0]
    enc = encoder_output.reshape(N, H)
    pv = pitch_target.reshape(N)
    ev = energy_target.reshape(N)
    ctab = jnp.concatenate([pitch_table, energy_table], axis=0)
    R = 2 * NBINS
    CG = H // CW
    ctab = (ctab.astype(jnp.bfloat16)
            .reshape(R, CG, CW // 32, 2, 16)
            .transpose(1, 0, 2, 4, 3)
            .reshape(CG, R, CW // 2, 2))
    ctab = jax.lax.bitcast_convert_type(ctab, jnp.int32)
    bnds = jnp.concatenate([
        jnp.linspace(50.0, 400.0, NBINS),
        jnp.linspace(0.0, 1.0, NBINS),
    ])
    out = _sc_call(N, H, NBINS)(enc, pv, ev, ctab, bnds)
    return out.reshape(B, S, H)
